# Initial kernel scaffold; baseline (speedup 1.0000x reference)
import jax
import jax.numpy as jnp
from jax.experimental import pallas as pl


def kernel(input_batch, probabilities, indices, W, b):
    raise NotImplementedError("write your pallas kernel here")


# speedup vs baseline: 2.4446x; 2.4446x over previous
import jax
import jax.numpy as jnp
from jax.experimental import pallas as pl

N_TOK = 2048
D = 1024
N_EXP = 8
TOPK = 2


def _moe_dense_body(idx_ref, p_ref, x_ref, w_ref, b_ref, out_ref):
    e = pl.program_id(0)

    @pl.when(e == 0)
    def _():
        out_ref[...] = jnp.zeros_like(out_ref)

    gate = jnp.sum(
        p_ref[...] * (idx_ref[...] == e).astype(jnp.float32), axis=1
    )
    y = jnp.dot(x_ref[...], w_ref[0], preferred_element_type=jnp.float32)
    y = y + b_ref[0]
    out_ref[...] += gate[:, None] * y


def kernel(input_batch, probabilities, indices, W, b):
    idx = indices.astype(jnp.int32)
    out = pl.pallas_call(
        _moe_dense_body,
        grid=(N_EXP,),
        in_specs=[
            pl.BlockSpec((N_TOK, TOPK), lambda e: (0, 0)),
            pl.BlockSpec((N_TOK, TOPK), lambda e: (0, 0)),
            pl.BlockSpec((N_TOK, D), lambda e: (0, 0)),
            pl.BlockSpec((1, D, D), lambda e: (e, 0, 0)),
            pl.BlockSpec((1, 1, D), lambda e: (e, 0, 0)),
        ],
        out_specs=pl.BlockSpec((N_TOK, D), lambda e: (0, 0)),
        out_shape=jax.ShapeDtypeStruct((N_TOK, D), jnp.float32),
    )(idx, probabilities, input_batch, W, b.reshape(N_EXP, 1, D))
    total_loss = jnp.zeros((), dtype=jnp.float32)
    return (out, total_loss)
